# Initial kernel scaffold; baseline (speedup 1.0000x reference)
import jax
import jax.numpy as jnp
from jax.experimental import pallas as pl


def kernel(u, nn_indices, dist_intp_coord_axis1, dist_intp_coord_axis2, weight_D, weight_c):
    raise NotImplementedError("write your pallas kernel here")


# speedup vs baseline: 8.5111x; 8.5111x over previous
import functools

import jax
import jax.numpy as jnp
from jax import lax
from jax.experimental import pallas as pl
from jax.experimental.pallas import tpu as pltpu
from jax.experimental.pallas import tpu_sc as plsc

P = 100000
NN = 8
T = 16
L = 16
C = 32
NW = 32

H = 0.01
EPS = 1e-8
INV_H_HALF = 0.5 / H
INV_H2 = 1.0 / (H * H)


def _make_sc_kernel(num_points, interpret=False):
  groups = C // L
  nchunks = num_points // C
  base_chunks = nchunks // NW
  extra = nchunks % NW

  def _sc_body(uT, idx2, d1f, d2f, Dh, ch, outT,
               D_v, idx_v, g_v, d1_v, d2_v, c_v, uo_v, out_v, sem):
    wid = lax.axis_index("s") * 2 + lax.axis_index("c")
    pltpu.sync_copy(Dh, D_v)
    nch = base_chunks + (wid < extra).astype(jnp.int32)
    iota = lax.iota(jnp.int32, L)

    def chunk_body(t, carry):
      chunk = wid + NW * t
      base = chunk * C
      pltpu.sync_copy(idx2.at[pl.ds(chunk * groups, groups)], idx_v)
      cps = [pltpu.async_copy(uT.at[idx_v.at[g]], g_v.at[g], sem)
             for g in range(groups)]
      pltpu.sync_copy(d1f.at[pl.ds(base * 3 * NN, C * 3 * NN)], d1_v)
      pltpu.sync_copy(d2f.at[pl.ds(base * 3 * NN, C * 3 * NN)], d2_v)
      pltpu.sync_copy(ch.at[pl.ds(base, C)], c_v)
      pltpu.sync_copy(uT.at[pl.ds(base, C)], uo_v)
      for cp in cps:
        cp.wait()

      for g in range(groups):
        plane = g * L + iota
        gfull = jnp.full((L,), g, jnp.int32)

        def axis_weights(dref):
          ws = []
          for i in range(3):
            r = [1.0 / (plsc.load_gather(
                    dref, [plane * (3 * NN) + (i * NN + n)]) + EPS)
                 for n in range(NN)]
            s = r[0]
            for n in range(1, NN):
              s = s + r[n]
            inv = 1.0 / s
            ws.append([x * inv for x in r])
          a = [(ws[2][n] - ws[0][n]) * INV_H_HALF for n in range(NN)]
          b = [(ws[0][n] - 2.0 * ws[1][n] + ws[2][n]) * INV_H2
               for n in range(NN)]
          return a, b

        a1, b1 = axis_weights(d1_v)
        a2, b2 = axis_weights(d2_v)

        j8 = [iota * NN + n for n in range(NN)]
        gi = [plsc.load_gather(idx_v, [gfull, j8[n]]) for n in range(NN)]
        Dg = [plsc.load_gather(D_v, [gi[n]]) for n in range(NN)]
        dD1 = a1[0] * Dg[0]
        dD2 = a2[0] * Dg[0]
        for n in range(1, NN):
          dD1 = dD1 + a1[n] * Dg[n]
          dD2 = dD2 + a2[n] * Dg[n]
        Down = plsc.load_gather(D_v, [base + plane])
        W = [dD1 * a1[n] + dD2 * a2[n] + Down * (b1[n] + b2[n])
             for n in range(NN)]
        cown = plsc.load_gather(c_v, [plane])

        for tt in range(T):
          tfull = jnp.full((L,), tt, jnp.int32)
          acc = cown * plsc.load_gather(uo_v, [plane, tfull])
          for n in range(NN):
            gv = plsc.load_gather(g_v, [gfull, j8[n], tfull])
            acc = acc + W[n] * gv
          plsc.store_scatter(out_v, [plane, tfull], acc)

      pltpu.sync_copy(out_v, outT.at[pl.ds(base, C)])
      return carry

    lax.fori_loop(0, nch, chunk_body, jnp.int32(0))

  return functools.partial(
      pl.kernel,
      out_type=jax.ShapeDtypeStruct((num_points, T), jnp.float32),
      mesh=plsc.VectorSubcoreMesh(core_axis_name="c", subcore_axis_name="s",
                                  num_cores=2, num_subcores=16),
      compiler_params=pltpu.CompilerParams(
          needs_layout_passes=False, use_tc_tiling_on_sc=False),
      interpret=interpret,
      scratch_types=[
          pltpu.VMEM((num_points,), jnp.float32),
          pltpu.VMEM((groups, 128), jnp.int32),
          pltpu.VMEM((groups, 128, T), jnp.float32),
          pltpu.VMEM((C * 3 * NN,), jnp.float32),
          pltpu.VMEM((C * 3 * NN,), jnp.float32),
          pltpu.VMEM((C,), jnp.float32),
          pltpu.VMEM((C, T), jnp.float32),
          pltpu.VMEM((C, T), jnp.float32),
          pltpu.SemaphoreType.DMA,
      ],
  )(_sc_body)


_sc_kernel = _make_sc_kernel(P)


def kernel(u, nn_indices, dist_intp_coord_axis1, dist_intp_coord_axis2,
           weight_D, weight_c):
  uT = u.T.astype(jnp.float32)
  idx2 = nn_indices.astype(jnp.int32).reshape(P * NN // 128, 128)
  d1f = dist_intp_coord_axis1[:, 3:6, :].reshape(P * 3 * NN)
  d2f = dist_intp_coord_axis2[:, 3:6, :].reshape(P * 3 * NN)
  D = weight_D.reshape(P).astype(jnp.float32)
  c = weight_c.reshape(P).astype(jnp.float32)
  outT = _sc_kernel(uT, idx2, d1f, d2f, D, c)
  return outT.T
